# baseline (speedup 1.0000x reference)
import functools

import jax
import jax.numpy as jnp
from jax.experimental import pallas as pl

P = 10000
F_IN = 16
T = 12
H = 64
E = 160000
T_OUT = 12

PB = 1000
EB = 1280


def _k1_proj(x_ref, wl_ref, wr_ref, xl_ref, xr_ref):
    x = x_ref[0]
    xl_ref[0] = x @ wl_ref[...]
    xr_ref[0] = x @ wr_ref[...]


def _k2_edge(ms_ref, md_ref, att_ref, ex_ref):
    m = ms_ref[0] + md_ref[0]
    lk = jnp.where(m > 0, m, 0.2 * m)
    e = jnp.sum(lk * att_ref[0][None, :], axis=-1, keepdims=True)
    ex_ref[0] = jnp.exp(e)


def _k3_weight(ex_ref, dend_ref, ms_ref, w_ref):
    alpha = ex_ref[...] / (dend_ref[...] + 1e-16)
    w_ref[...] = alpha * ms_ref[...]


def _k4_tail(gat_ref, x_ref, bgat_ref, wt_ref, bt_ref, wres_ref, bres_ref,
             gamma_ref, beta_ref, wf_ref, bf_ref, out_ref):
    bgat = bgat_ref[0][None, :]
    bt = bt_ref[0][None, :]
    bres = bres_ref[0][None, :]
    gamma = gamma_ref[0][None, :]
    beta = beta_ref[0][None, :]

    g = [jnp.maximum(gat_ref[:, t, :] + bgat, 0.0) for t in range(T)]

    w0 = wt_ref[0]
    w1 = wt_ref[1]
    w2 = wt_ref[2]
    wres = wres_ref[...]

    acc = jnp.zeros((out_ref.shape[0], T_OUT), dtype=jnp.float32)
    for t in range(T):
        y = g[t] @ w1
        if t - 1 >= 0:
            y = y + g[t - 1] @ w0
        if t + 1 < T:
            y = y + g[t + 1] @ w2
        xres = x_ref[:, t, :] @ wres
        z = jnp.maximum(y + bt + xres + bres, 0.0)
        mu = jnp.mean(z, axis=-1, keepdims=True)
        var = jnp.mean((z - mu) * (z - mu), axis=-1, keepdims=True)
        xn = (z - mu) * jax.lax.rsqrt(var + 1e-5) * gamma + beta
        acc = acc + xn @ wf_ref[t]
    out_ref[...] = acc + bf_ref[0][None, :]


def kernel(X, edge_index, Wl, Wr, att, b_gat, Wt, bt, Wres, bres, gamma, beta, Wf, bf):
    src = edge_index[:, 0, :]
    dst = edge_index[:, 1, :]

    XT = jnp.transpose(X[0], (2, 0, 1))

    XL, XR = pl.pallas_call(
        _k1_proj,
        grid=(T, P // PB),
        in_specs=[
            pl.BlockSpec((1, PB, F_IN), lambda t, p: (t, p, 0)),
            pl.BlockSpec((F_IN, H), lambda t, p: (0, 0)),
            pl.BlockSpec((F_IN, H), lambda t, p: (0, 0)),
        ],
        out_specs=[
            pl.BlockSpec((1, PB, H), lambda t, p: (t, p, 0)),
            pl.BlockSpec((1, PB, H), lambda t, p: (t, p, 0)),
        ],
        out_shape=[
            jax.ShapeDtypeStruct((T, P, H), jnp.float32),
            jax.ShapeDtypeStruct((T, P, H), jnp.float32),
        ],
    )(XT, Wl, Wr)

    ms = jnp.take_along_axis(XL, src[:, :, None], axis=1)
    md = jnp.take_along_axis(XR, dst[:, :, None], axis=1)

    att2 = att[None, :]
    ex = pl.pallas_call(
        _k2_edge,
        grid=(T, E // EB),
        in_specs=[
            pl.BlockSpec((1, EB, H), lambda t, e: (t, e, 0)),
            pl.BlockSpec((1, EB, H), lambda t, e: (t, e, 0)),
            pl.BlockSpec((1, H), lambda t, e: (0, 0)),
        ],
        out_specs=pl.BlockSpec((1, EB, 1), lambda t, e: (t, e, 0)),
        out_shape=jax.ShapeDtypeStruct((T, E, 1), jnp.float32),
    )(ms, md, att2)

    seg_sum = jax.vmap(
        functools.partial(jax.ops.segment_sum, num_segments=P))
    den = seg_sum(ex[..., 0], dst)
    dend = jnp.take_along_axis(den, dst, axis=1)[..., None]

    w = pl.pallas_call(
        _k3_weight,
        grid=(T, E // EB),
        in_specs=[
            pl.BlockSpec((1, EB, 1), lambda t, e: (t, e, 0)),
            pl.BlockSpec((1, EB, 1), lambda t, e: (t, e, 0)),
            pl.BlockSpec((1, EB, H), lambda t, e: (t, e, 0)),
        ],
        out_specs=pl.BlockSpec((1, EB, H), lambda t, e: (t, e, 0)),
        out_shape=jax.ShapeDtypeStruct((T, E, H), jnp.float32),
    )(ex, dend, ms)

    out_gat = seg_sum(w, dst)

    gat_p = jnp.transpose(out_gat, (1, 0, 2))
    Xp = jnp.transpose(X[0], (0, 2, 1))
    Wt_r = jnp.transpose(Wt[:, :, 0, :], (2, 1, 0))
    WresT = jnp.transpose(Wres[:, :, 0, 0], (1, 0))
    WfT = jnp.transpose(Wf[:, :, 0, :], (1, 2, 0))

    out = pl.pallas_call(
        _k4_tail,
        grid=(P // PB,),
        in_specs=[
            pl.BlockSpec((PB, T, H), lambda p: (p, 0, 0)),
            pl.BlockSpec((PB, T, F_IN), lambda p: (p, 0, 0)),
            pl.BlockSpec((1, H), lambda p: (0, 0)),
            pl.BlockSpec((3, H, H), lambda p: (0, 0, 0)),
            pl.BlockSpec((1, H), lambda p: (0, 0)),
            pl.BlockSpec((F_IN, H), lambda p: (0, 0)),
            pl.BlockSpec((1, H), lambda p: (0, 0)),
            pl.BlockSpec((1, H), lambda p: (0, 0)),
            pl.BlockSpec((1, H), lambda p: (0, 0)),
            pl.BlockSpec((T, H, T_OUT), lambda p: (0, 0, 0)),
            pl.BlockSpec((1, T_OUT), lambda p: (0, 0)),
        ],
        out_specs=pl.BlockSpec((PB, T_OUT), lambda p: (p, 0)),
        out_shape=jax.ShapeDtypeStruct((P, T_OUT), jnp.float32),
    )(gat_p, Xp, b_gat[None, :], Wt_r, bt[None, :], WresT, bres[None, :],
      gamma[None, :], beta[None, :], WfT, bf[None, :])

    return out[None]
